# baseline (speedup 1.0000x reference)
import functools

import jax
import jax.numpy as jnp
from jax import lax
from jax.experimental import pallas as pl
from jax.experimental.pallas import tpu as pltpu
from jax.experimental.pallas import tpu_sc as plsc

_B = 4096
_D_IN = 1024
_M = 32
_D = 64
_K = 1024
_H = 64
_BETA = 0.25

_BB_ENC = 512
_FB = 1024
_BB_HEAD = 256

_NC = 2
_NS = 16
_NW = _NC * _NS
_GCHUNK = 512


def _enc_kernel(f_ref, w_ref, b_ref, o_ref):
    o_ref[...] = (
        jnp.dot(f_ref[...], w_ref[...], preferred_element_type=jnp.float32)
        + b_ref[...]
    )


def _vq_kernel(flat_ref, cbt_ref, idx_ref, cnt_ref, loss_ref):
    g = pl.program_id(0)

    flat = flat_ref[...]
    cbt = cbt_ref[...]
    cb_sq = jnp.sum(cbt * cbt, axis=0, keepdims=True)
    row_sq = jnp.sum(flat * flat, axis=1, keepdims=True)
    scores = (row_sq + cb_sq) - 2.0 * jnp.dot(
        flat, cbt, preferred_element_type=jnp.float32
    )

    rowmin = jnp.min(scores, axis=1, keepdims=True)
    iota = lax.broadcasted_iota(jnp.int32, (_FB, _K), 1)
    idx = jnp.min(
        jnp.where(scores == rowmin, iota, _K), axis=1, keepdims=True
    )
    idx_ref[0, 0, :] = idx[:, 0]

    onehot = (iota == idx).astype(jnp.float32)

    @pl.when(g == 0)
    def _init():
        cnt_ref[...] = jnp.zeros_like(cnt_ref)
        loss_ref[...] = jnp.zeros_like(loss_ref)

    cnt_ref[0:1, :] += jnp.sum(onehot, axis=0, keepdims=True)
    loss_ref[...] += jnp.sum(rowmin, axis=0, keepdims=True)


def _head_kernel(zq_ref, w1_ref, b1_ref, w2_ref, b2_ref, wd_ref, bd_ref,
                 cnt_ref, loss_ref, logits_ref, fhat_ref, vql_ref, perp_ref):
    i = pl.program_id(0)
    zq = zq_ref[...]
    h = jax.nn.gelu(
        jnp.dot(zq, w1_ref[...], preferred_element_type=jnp.float32)
        + b1_ref[...]
    )
    logits_ref[...] = (
        jnp.dot(h, w2_ref[...], preferred_element_type=jnp.float32)
        + b2_ref[...]
    )
    fhat_ref[...] = (
        jnp.dot(zq, wd_ref[...], preferred_element_type=jnp.float32)
        + bd_ref[...]
    )

    @pl.when(i == 0)
    def _scalars():
        counts = cnt_ref[0:1, :]
        total = jnp.sum(counts, axis=1, keepdims=True)
        probs = counts / total
        ent = jnp.sum(probs * jnp.log(probs + 1e-10), axis=1, keepdims=True)
        perp_ref[...] = jnp.exp(-ent)
        vql_ref[...] = loss_ref[...] * ((1.0 + _BETA) / (_B * _M * _D))


def _sc_gather(codebook_pad, idx_flat):
    n = idx_flat.shape[0]
    per_w = n // _NW
    mesh = plsc.VectorSubcoreMesh(core_axis_name="c", subcore_axis_name="s")

    @functools.partial(
        pl.kernel,
        mesh=mesh,
        out_type=jax.ShapeDtypeStruct((n, 128), jnp.float32),
        scratch_types=[
            pltpu.VMEM((_GCHUNK,), jnp.int32),
            pltpu.VMEM((_GCHUNK, 128), jnp.float32),
            pltpu.SemaphoreType.DMA,
        ],
    )
    def k(table_hbm, idx_hbm, out_hbm, idx_v, rows_v, sem):
        wid = lax.axis_index("s") * _NC + lax.axis_index("c")
        base = wid * per_w

        @pl.loop(0, per_w, step=_GCHUNK)
        def _(off):
            pltpu.sync_copy(idx_hbm.at[pl.ds(base + off, _GCHUNK)], idx_v)
            pltpu.async_copy(table_hbm.at[idx_v], rows_v, sem).wait()
            pltpu.sync_copy(rows_v, out_hbm.at[pl.ds(base + off, _GCHUNK)])

    return k(codebook_pad, idx_flat)


def _depad_kernel(xp_ref, o_ref):
    o_ref[...] = xp_ref[:, : _D]


def _depad(zq_pad):
    n = zq_pad.shape[0]
    fb = 8192
    return pl.pallas_call(
        _depad_kernel,
        grid=(n // fb,),
        in_specs=[pl.BlockSpec((fb, 128), lambda i: (i, 0))],
        out_specs=pl.BlockSpec((fb, _D), lambda i: (i, 0)),
        out_shape=jax.ShapeDtypeStruct((n, _D), jnp.float32),
    )(zq_pad)


@jax.jit
def kernel(f_t, W_enc, b_enc, codebook, W1, b1, W2, b2, W_dec, b_dec):
    z_e = pl.pallas_call(
        _enc_kernel,
        grid=(_B // _BB_ENC,),
        in_specs=[
            pl.BlockSpec((_BB_ENC, _D_IN), lambda i: (i, 0)),
            pl.BlockSpec((_D_IN, _M * _D), lambda i: (0, 0)),
            pl.BlockSpec((1, _M * _D), lambda i: (0, 0)),
        ],
        out_specs=pl.BlockSpec((_BB_ENC, _M * _D), lambda i: (i, 0)),
        out_shape=jax.ShapeDtypeStruct((_B, _M * _D), jnp.float32),
    )(f_t, W_enc, b_enc.reshape(1, _M * _D))

    flat = z_e.reshape(_B * _M, _D)
    n_vq = _B * _M // _FB

    idx_blocks, counts8, loss_sum = pl.pallas_call(
        _vq_kernel,
        grid=(n_vq,),
        in_specs=[
            pl.BlockSpec((_FB, _D), lambda g: (g, 0)),
            pl.BlockSpec((_D, _K), lambda g: (0, 0)),
        ],
        out_specs=[
            pl.BlockSpec((1, 1, _FB), lambda g: (g, 0, 0)),
            pl.BlockSpec((8, _K), lambda g: (0, 0)),
            pl.BlockSpec((1, 1), lambda g: (0, 0)),
        ],
        out_shape=[
            jax.ShapeDtypeStruct((n_vq, 1, _FB), jnp.int32),
            jax.ShapeDtypeStruct((8, _K), jnp.float32),
            jax.ShapeDtypeStruct((1, 1), jnp.float32),
        ],
    )(flat, codebook.T)

    idx_flat = idx_blocks.reshape(_B * _M)

    cb_pad = jnp.pad(codebook, ((0, 0), (0, 128 - _D)))
    zq_pad = _sc_gather(cb_pad, idx_flat)
    zq_flat = _depad(zq_pad)

    z_q2 = zq_flat.reshape(_B, _M * _D)
    W2p = jnp.pad(W2, ((0, 0), (0, 128 - W2.shape[1])))
    b2p = jnp.pad(b2, (0, 128 - b2.shape[0])).reshape(1, 128)

    logits_p, f_hat, vql, perp = pl.pallas_call(
        _head_kernel,
        grid=(_B // _BB_HEAD,),
        in_specs=[
            pl.BlockSpec((_BB_HEAD, _M * _D), lambda i: (i, 0)),
            pl.BlockSpec((_M * _D, _H), lambda i: (0, 0)),
            pl.BlockSpec((1, _H), lambda i: (0, 0)),
            pl.BlockSpec((_H, 128), lambda i: (0, 0)),
            pl.BlockSpec((1, 128), lambda i: (0, 0)),
            pl.BlockSpec((_M * _D, _D_IN), lambda i: (0, 0)),
            pl.BlockSpec((1, _D_IN), lambda i: (0, 0)),
            pl.BlockSpec((8, _K), lambda i: (0, 0)),
            pl.BlockSpec((1, 1), lambda i: (0, 0)),
        ],
        out_specs=[
            pl.BlockSpec((_BB_HEAD, 128), lambda i: (i, 0)),
            pl.BlockSpec((_BB_HEAD, _D_IN), lambda i: (i, 0)),
            pl.BlockSpec((1, 1), lambda i: (0, 0)),
            pl.BlockSpec((1, 1), lambda i: (0, 0)),
        ],
        out_shape=[
            jax.ShapeDtypeStruct((_B, 128), jnp.float32),
            jax.ShapeDtypeStruct((_B, _D_IN), jnp.float32),
            jax.ShapeDtypeStruct((1, 1), jnp.float32),
            jax.ShapeDtypeStruct((1, 1), jnp.float32),
        ],
    )(z_q2, W1, b1.reshape(1, _H), W2p, b2p, W_dec, b_dec.reshape(1, _D_IN),
      counts8, loss_sum)

    logits = logits_p[:, : W2.shape[1]]
    z_q = zq_flat.reshape(_B, _M, _D)
    indices = idx_blocks.reshape(_B, _M)
    return (logits, f_hat, z_q, indices,
            vql.reshape(()), perp.reshape(()))


# speedup vs baseline: 1.6299x; 1.6299x over previous
import functools

import jax
import jax.numpy as jnp
from jax import lax
from jax.experimental import pallas as pl
from jax.experimental.pallas import tpu as pltpu
from jax.experimental.pallas import tpu_sc as plsc

_B = 4096
_D_IN = 1024
_M = 32
_D = 64
_K = 1024
_H = 64
_BETA = 0.25

_BB_VQ = 256
_BB_HEAD = 256

_NC = 2
_NS = 16
_NW = _NC * _NS
_GCHUNK = 256


def _encvq_kernel(f_ref, w_ref, b_ref, cbt2_ref, idx_ref, cnt_ref, loss_ref):
    g = pl.program_id(0)

    z_e = (
        jnp.dot(f_ref[...], w_ref[...], preferred_element_type=jnp.float32)
        + b_ref[...]
    )
    cbt2 = cbt2_ref[...]
    cb2_sq = jnp.sum(cbt2 * cbt2, axis=0, keepdims=True)

    iota_f = lax.broadcasted_iota(
        jnp.int32, (_BB_VQ, _K), 1
    ).astype(jnp.float32)
    big = jnp.float32(_K)

    idx_cols = []
    cnt_acc = jnp.zeros((1, _K), jnp.float32)
    loss_acc = jnp.zeros((1, 1), jnp.float32)
    for c in range(_M // 2):
        e_pair = z_e[:, 128 * c : 128 * (c + 1)]
        pair_scores = cb2_sq - 2.0 * jnp.dot(
            e_pair, cbt2, preferred_element_type=jnp.float32
        )
        for h in range(2):
            s = pair_scores[:, _K * h : _K * (h + 1)]
            rowmin = jnp.min(s, axis=1, keepdims=True)
            cand = jnp.where(s == rowmin, iota_f, big)
            idxf = jnp.min(cand, axis=1, keepdims=True)
            onehot = (cand == idxf).astype(jnp.float32)
            cnt_acc = cnt_acc + jnp.sum(onehot, axis=0, keepdims=True)
            loss_acc = loss_acc + jnp.sum(rowmin, axis=0, keepdims=True)
            idx_cols.append(idxf)

    idx_ref[...] = jnp.concatenate(idx_cols, axis=1).astype(jnp.int32)

    loss_acc = loss_acc + jnp.sum(
        jnp.sum(z_e * z_e, axis=1, keepdims=True), axis=0, keepdims=True
    )

    @pl.when(g == 0)
    def _init():
        cnt_ref[...] = jnp.zeros_like(cnt_ref)
        loss_ref[...] = jnp.zeros_like(loss_ref)

    cnt_ref[0:1, :] += cnt_acc
    loss_ref[...] += loss_acc


def _head_kernel(zq_ref, w1_ref, b1_ref, w2_ref, b2_ref, wd_ref, bd_ref,
                 cnt_ref, loss_ref, logits_ref, fhat_ref, vql_ref, perp_ref):
    i = pl.program_id(0)
    zq = zq_ref[...]
    h = jax.nn.gelu(
        jnp.dot(zq, w1_ref[...], preferred_element_type=jnp.float32)
        + b1_ref[...]
    )
    logits_ref[...] = (
        jnp.dot(h, w2_ref[...], preferred_element_type=jnp.float32)
        + b2_ref[...]
    )
    fhat_ref[...] = (
        jnp.dot(zq, wd_ref[...], preferred_element_type=jnp.float32)
        + bd_ref[...]
    )

    @pl.when(i == 0)
    def _scalars():
        counts = cnt_ref[0:1, :]
        total = jnp.sum(counts, axis=1, keepdims=True)
        probs = counts / total
        ent = jnp.sum(probs * jnp.log(probs + 1e-10), axis=1, keepdims=True)
        perp_ref[...] = jnp.exp(-ent)
        vql_ref[...] = loss_ref[...] * ((1.0 + _BETA) / (_B * _M * _D))


def _sc_gather(codebook_pad, idx_flat):
    n = idx_flat.shape[0]
    per_w = n // _NW
    mesh = plsc.VectorSubcoreMesh(core_axis_name="c", subcore_axis_name="s")

    n_it = per_w // _GCHUNK

    @functools.partial(
        pl.kernel,
        mesh=mesh,
        out_type=jax.ShapeDtypeStruct((n, 128), jnp.float32),
        scratch_types=[
            pltpu.VMEM((_GCHUNK,), jnp.int32),
            pltpu.VMEM((_GCHUNK,), jnp.int32),
            pltpu.VMEM((_GCHUNK, 128), jnp.float32),
            pltpu.VMEM((_GCHUNK, 128), jnp.float32),
            pltpu.SemaphoreType.DMA,
            pltpu.SemaphoreType.DMA,
            pltpu.SemaphoreType.DMA,
            pltpu.SemaphoreType.DMA,
        ],
    )
    def k(table_hbm, idx_hbm, out_hbm, idx_v0, idx_v1, rows_v0, rows_v1,
          gs0, gs1, ws0, ws1):
        wid = lax.axis_index("s") * _NC + lax.axis_index("c")
        base = wid * per_w
        idx_v = (idx_v0, idx_v1)
        rows_v = (rows_v0, rows_v1)
        gs = (gs0, gs1)
        ws = (ws0, ws1)

        g_h = [None] * n_it
        w_h = [None] * n_it
        pltpu.sync_copy(idx_hbm.at[pl.ds(base, _GCHUNK)], idx_v[0])
        g_h[0] = pltpu.async_copy(table_hbm.at[idx_v[0]], rows_v[0], gs[0])
        for j in range(n_it):
            b = j % 2
            nb = (j + 1) % 2
            if j + 1 < n_it:
                if j >= 1:
                    w_h[j - 1].wait()
                pltpu.sync_copy(
                    idx_hbm.at[pl.ds(base + (j + 1) * _GCHUNK, _GCHUNK)],
                    idx_v[nb],
                )
                g_h[j + 1] = pltpu.async_copy(
                    table_hbm.at[idx_v[nb]], rows_v[nb], gs[nb]
                )
            g_h[j].wait()
            w_h[j] = pltpu.async_copy(
                rows_v[b], out_hbm.at[pl.ds(base + j * _GCHUNK, _GCHUNK)],
                ws[b],
            )
        w_h[n_it - 2].wait()
        w_h[n_it - 1].wait()

    return k(codebook_pad, idx_flat)


def _depad_kernel(xp_ref, o_ref):
    o_ref[...] = xp_ref[:, : _D]


def _depad(zq_pad):
    n = zq_pad.shape[0]
    fb = 8192
    return pl.pallas_call(
        _depad_kernel,
        grid=(n // fb,),
        in_specs=[pl.BlockSpec((fb, 128), lambda i: (i, 0))],
        out_specs=pl.BlockSpec((fb, _D), lambda i: (i, 0)),
        out_shape=jax.ShapeDtypeStruct((n, _D), jnp.float32),
    )(zq_pad)


@jax.jit
def kernel(f_t, W_enc, b_enc, codebook, W1, b1, W2, b2, W_dec, b_dec):
    cbt = codebook.T
    cbt2 = jnp.zeros((2 * _D, 2 * _K), jnp.float32)
    cbt2 = cbt2.at[: _D, : _K].set(cbt).at[_D :, _K :].set(cbt)

    indices, counts8, loss_sum = pl.pallas_call(
        _encvq_kernel,
        grid=(_B // _BB_VQ,),
        in_specs=[
            pl.BlockSpec((_BB_VQ, _D_IN), lambda g: (g, 0)),
            pl.BlockSpec((_D_IN, _M * _D), lambda g: (0, 0)),
            pl.BlockSpec((1, _M * _D), lambda g: (0, 0)),
            pl.BlockSpec((2 * _D, 2 * _K), lambda g: (0, 0)),
        ],
        out_specs=[
            pl.BlockSpec((_BB_VQ, _M), lambda g: (g, 0)),
            pl.BlockSpec((8, _K), lambda g: (0, 0)),
            pl.BlockSpec((1, 1), lambda g: (0, 0)),
        ],
        out_shape=[
            jax.ShapeDtypeStruct((_B, _M), jnp.int32),
            jax.ShapeDtypeStruct((8, _K), jnp.float32),
            jax.ShapeDtypeStruct((1, 1), jnp.float32),
        ],
    )(f_t, W_enc, b_enc.reshape(1, _M * _D), cbt2)

    idx_flat = indices.reshape(_B * _M)

    cb_pad = jnp.pad(codebook, ((0, 0), (0, 128 - _D)))
    zq_pad = _sc_gather(cb_pad, idx_flat)
    zq_flat = _depad(zq_pad)

    z_q2 = zq_flat.reshape(_B, _M * _D)
    W2p = jnp.pad(W2, ((0, 0), (0, 128 - W2.shape[1])))
    b2p = jnp.pad(b2, (0, 128 - b2.shape[0])).reshape(1, 128)

    logits_p, f_hat, vql, perp = pl.pallas_call(
        _head_kernel,
        grid=(_B // _BB_HEAD,),
        in_specs=[
            pl.BlockSpec((_BB_HEAD, _M * _D), lambda i: (i, 0)),
            pl.BlockSpec((_M * _D, _H), lambda i: (0, 0)),
            pl.BlockSpec((1, _H), lambda i: (0, 0)),
            pl.BlockSpec((_H, 128), lambda i: (0, 0)),
            pl.BlockSpec((1, 128), lambda i: (0, 0)),
            pl.BlockSpec((_M * _D, _D_IN), lambda i: (0, 0)),
            pl.BlockSpec((1, _D_IN), lambda i: (0, 0)),
            pl.BlockSpec((8, _K), lambda i: (0, 0)),
            pl.BlockSpec((1, 1), lambda i: (0, 0)),
        ],
        out_specs=[
            pl.BlockSpec((_BB_HEAD, 128), lambda i: (i, 0)),
            pl.BlockSpec((_BB_HEAD, _D_IN), lambda i: (i, 0)),
            pl.BlockSpec((1, 1), lambda i: (0, 0)),
            pl.BlockSpec((1, 1), lambda i: (0, 0)),
        ],
        out_shape=[
            jax.ShapeDtypeStruct((_B, 128), jnp.float32),
            jax.ShapeDtypeStruct((_B, _D_IN), jnp.float32),
            jax.ShapeDtypeStruct((1, 1), jnp.float32),
            jax.ShapeDtypeStruct((1, 1), jnp.float32),
        ],
    )(z_q2, W1, b1.reshape(1, _H), W2p, b2p, W_dec, b_dec.reshape(1, _D_IN),
      counts8, loss_sum)

    logits = logits_p[:, : W2.shape[1]]
    z_q = zq_flat.reshape(_B, _M, _D)
    return (logits, f_hat, z_q, indices,
            vql.reshape(()), perp.reshape(()))


# speedup vs baseline: 1.6365x; 1.0041x over previous
import functools

import jax
import jax.numpy as jnp
from jax import lax
from jax.experimental import pallas as pl
from jax.experimental.pallas import tpu as pltpu
from jax.experimental.pallas import tpu_sc as plsc

_B = 4096
_D_IN = 1024
_M = 32
_D = 64
_K = 1024
_H = 64
_BETA = 0.25

_BB_VQ = 256
_BB_HEAD = 512

_NC = 2
_NS = 16
_NW = _NC * _NS
_GCHUNK = 256


def _encvq_kernel(f_ref, w_ref, b_ref, cbt2_ref, idx_ref, cnt_ref, loss_ref):
    g = pl.program_id(0)

    z_e = (
        jnp.dot(f_ref[...], w_ref[...], preferred_element_type=jnp.float32)
        + b_ref[...]
    )
    cbt2 = cbt2_ref[...]
    cb2_sq = jnp.sum(cbt2 * cbt2, axis=0, keepdims=True)

    iota_f = lax.broadcasted_iota(
        jnp.int32, (_BB_VQ, _K), 1
    ).astype(jnp.float32)
    big = jnp.float32(_K)

    idx_cols = []
    cnt_acc = jnp.zeros((1, _K), jnp.float32)
    loss_acc = jnp.zeros((1, 1), jnp.float32)
    for c in range(_M // 2):
        e_pair = z_e[:, 128 * c : 128 * (c + 1)]
        pair_scores = cb2_sq - 2.0 * jnp.dot(
            e_pair, cbt2, preferred_element_type=jnp.float32
        )
        for h in range(2):
            s = pair_scores[:, _K * h : _K * (h + 1)]
            rowmin = jnp.min(s, axis=1, keepdims=True)
            cand = jnp.where(s == rowmin, iota_f, big)
            idxf = jnp.min(cand, axis=1, keepdims=True)
            onehot = (cand == idxf).astype(jnp.float32)
            cnt_acc = cnt_acc + jnp.sum(onehot, axis=0, keepdims=True)
            loss_acc = loss_acc + jnp.sum(rowmin, axis=0, keepdims=True)
            idx_cols.append(idxf)

    idx_ref[...] = jnp.concatenate(idx_cols, axis=1).astype(jnp.int32)

    loss_acc = loss_acc + jnp.sum(
        jnp.sum(z_e * z_e, axis=1, keepdims=True), axis=0, keepdims=True
    )

    @pl.when(g == 0)
    def _init():
        cnt_ref[...] = jnp.zeros_like(cnt_ref)
        loss_ref[...] = jnp.zeros_like(loss_ref)

    cnt_ref[0:1, :] += cnt_acc
    loss_ref[...] += loss_acc


def _head_kernel(zq_ref, w1_ref, b1_ref, w2_ref, b2_ref, wd_ref, bd_ref,
                 cnt_ref, loss_ref, logits_ref, fhat_ref, vql_ref, perp_ref):
    i = pl.program_id(0)
    zq = zq_ref[...]
    h = jax.nn.gelu(
        jnp.dot(zq, w1_ref[...], preferred_element_type=jnp.float32)
        + b1_ref[...]
    )
    logits_ref[...] = (
        jnp.dot(h, w2_ref[...], preferred_element_type=jnp.float32)
        + b2_ref[...]
    )
    fhat_ref[...] = (
        jnp.dot(zq, wd_ref[...], preferred_element_type=jnp.float32)
        + bd_ref[...]
    )

    @pl.when(i == 0)
    def _scalars():
        counts = cnt_ref[0:1, :]
        total = jnp.sum(counts, axis=1, keepdims=True)
        probs = counts / total
        ent = jnp.sum(probs * jnp.log(probs + 1e-10), axis=1, keepdims=True)
        perp_ref[...] = jnp.exp(-ent)
        vql_ref[...] = loss_ref[...] * ((1.0 + _BETA) / (_B * _M * _D))


def _sc_gather(codebook_pad, idx_flat):
    n = idx_flat.shape[0]
    per_w = n // _NW
    mesh = plsc.VectorSubcoreMesh(core_axis_name="c", subcore_axis_name="s")

    n_it = per_w // _GCHUNK

    @functools.partial(
        pl.kernel,
        mesh=mesh,
        out_type=jax.ShapeDtypeStruct((n, 128), jnp.float32),
        scratch_types=[
            pltpu.VMEM((_GCHUNK,), jnp.int32),
            pltpu.VMEM((_GCHUNK,), jnp.int32),
            pltpu.VMEM((_GCHUNK, 128), jnp.float32),
            pltpu.VMEM((_GCHUNK, 128), jnp.float32),
            pltpu.SemaphoreType.DMA,
            pltpu.SemaphoreType.DMA,
            pltpu.SemaphoreType.DMA,
            pltpu.SemaphoreType.DMA,
        ],
    )
    def k(table_hbm, idx_hbm, out_hbm, idx_v0, idx_v1, rows_v0, rows_v1,
          gs0, gs1, ws0, ws1):
        wid = lax.axis_index("s") * _NC + lax.axis_index("c")
        base = wid * per_w
        idx_v = (idx_v0, idx_v1)
        rows_v = (rows_v0, rows_v1)
        gs = (gs0, gs1)
        ws = (ws0, ws1)

        g_h = [None] * n_it
        w_h = [None] * n_it
        pltpu.sync_copy(idx_hbm.at[pl.ds(base, _GCHUNK)], idx_v[0])
        g_h[0] = pltpu.async_copy(table_hbm.at[idx_v[0]], rows_v[0], gs[0])
        for j in range(n_it):
            b = j % 2
            nb = (j + 1) % 2
            if j + 1 < n_it:
                if j >= 1:
                    w_h[j - 1].wait()
                pltpu.sync_copy(
                    idx_hbm.at[pl.ds(base + (j + 1) * _GCHUNK, _GCHUNK)],
                    idx_v[nb],
                )
                g_h[j + 1] = pltpu.async_copy(
                    table_hbm.at[idx_v[nb]], rows_v[nb], gs[nb]
                )
            g_h[j].wait()
            w_h[j] = pltpu.async_copy(
                rows_v[b], out_hbm.at[pl.ds(base + j * _GCHUNK, _GCHUNK)],
                ws[b],
            )
        w_h[n_it - 2].wait()
        w_h[n_it - 1].wait()

    return k(codebook_pad, idx_flat)


def _depad_kernel(xp_ref, o_ref):
    o_ref[...] = xp_ref[:, : _D]


def _depad(zq_pad):
    n = zq_pad.shape[0]
    fb = 16384
    return pl.pallas_call(
        _depad_kernel,
        grid=(n // fb,),
        in_specs=[pl.BlockSpec((fb, 128), lambda i: (i, 0))],
        out_specs=pl.BlockSpec((fb, _D), lambda i: (i, 0)),
        out_shape=jax.ShapeDtypeStruct((n, _D), jnp.float32),
    )(zq_pad)


@jax.jit
def kernel(f_t, W_enc, b_enc, codebook, W1, b1, W2, b2, W_dec, b_dec):
    cbt = codebook.T
    cbt2 = jnp.zeros((2 * _D, 2 * _K), jnp.float32)
    cbt2 = cbt2.at[: _D, : _K].set(cbt).at[_D :, _K :].set(cbt)

    indices, counts8, loss_sum = pl.pallas_call(
        _encvq_kernel,
        grid=(_B // _BB_VQ,),
        in_specs=[
            pl.BlockSpec((_BB_VQ, _D_IN), lambda g: (g, 0)),
            pl.BlockSpec((_D_IN, _M * _D), lambda g: (0, 0)),
            pl.BlockSpec((1, _M * _D), lambda g: (0, 0)),
            pl.BlockSpec((2 * _D, 2 * _K), lambda g: (0, 0)),
        ],
        out_specs=[
            pl.BlockSpec((_BB_VQ, _M), lambda g: (g, 0)),
            pl.BlockSpec((8, _K), lambda g: (0, 0)),
            pl.BlockSpec((1, 1), lambda g: (0, 0)),
        ],
        out_shape=[
            jax.ShapeDtypeStruct((_B, _M), jnp.int32),
            jax.ShapeDtypeStruct((8, _K), jnp.float32),
            jax.ShapeDtypeStruct((1, 1), jnp.float32),
        ],
    )(f_t, W_enc, b_enc.reshape(1, _M * _D), cbt2)

    idx_flat = indices.reshape(_B * _M)

    cb_pad = jnp.pad(codebook, ((0, 0), (0, 128 - _D)))
    zq_pad = _sc_gather(cb_pad, idx_flat)
    zq_flat = _depad(zq_pad)

    z_q2 = zq_flat.reshape(_B, _M * _D)
    W2p = jnp.pad(W2, ((0, 0), (0, 128 - W2.shape[1])))
    b2p = jnp.pad(b2, (0, 128 - b2.shape[0])).reshape(1, 128)

    logits_p, f_hat, vql, perp = pl.pallas_call(
        _head_kernel,
        grid=(_B // _BB_HEAD,),
        in_specs=[
            pl.BlockSpec((_BB_HEAD, _M * _D), lambda i: (i, 0)),
            pl.BlockSpec((_M * _D, _H), lambda i: (0, 0)),
            pl.BlockSpec((1, _H), lambda i: (0, 0)),
            pl.BlockSpec((_H, 128), lambda i: (0, 0)),
            pl.BlockSpec((1, 128), lambda i: (0, 0)),
            pl.BlockSpec((_M * _D, _D_IN), lambda i: (0, 0)),
            pl.BlockSpec((1, _D_IN), lambda i: (0, 0)),
            pl.BlockSpec((8, _K), lambda i: (0, 0)),
            pl.BlockSpec((1, 1), lambda i: (0, 0)),
        ],
        out_specs=[
            pl.BlockSpec((_BB_HEAD, 128), lambda i: (i, 0)),
            pl.BlockSpec((_BB_HEAD, _D_IN), lambda i: (i, 0)),
            pl.BlockSpec((1, 1), lambda i: (0, 0)),
            pl.BlockSpec((1, 1), lambda i: (0, 0)),
        ],
        out_shape=[
            jax.ShapeDtypeStruct((_B, 128), jnp.float32),
            jax.ShapeDtypeStruct((_B, _D_IN), jnp.float32),
            jax.ShapeDtypeStruct((1, 1), jnp.float32),
            jax.ShapeDtypeStruct((1, 1), jnp.float32),
        ],
    )(z_q2, W1, b1.reshape(1, _H), W2p, b2p, W_dec, b_dec.reshape(1, _D_IN),
      counts8, loss_sum)

    logits = logits_p[:, : W2.shape[1]]
    z_q = zq_flat.reshape(_B, _M, _D)
    return (logits, f_hat, z_q, indices,
            vql.reshape(()), perp.reshape(()))
